# baseline (speedup 1.0000x reference)
import jax
import jax.numpy as jnp
from jax import lax
from jax.experimental import pallas as pl
from jax.experimental.pallas import tpu as pltpu
from jax.experimental.pallas import tpu_sc as plsc

W = 121
L = 16
NC, NS = 2, 16
NW = NC * NS

B, K = 16, 17
NBLK = B * K

YSPLITS = ((0, 31), (31, 30), (61, 30), (91, 30))
NSPLIT = len(YSPLITS)


def _body(f0_hbm, f1_hbm, flip_hbm, o0_hbm, o1_hbm,
          flip_v, i0, o0, i1, o1, *sems):
  isems = sems[:NSPLIT]
  osems = sems[NSPLIT:]
  wid = lax.axis_index("s") * NC + lax.axis_index("c")
  pltpu.sync_copy(flip_hbm, flip_v)

  def do_field(in_hbm, out_hbm, ibuf, obuf, c_dim, signed):
    nb = (NBLK - wid + NW - 1) // NW

    def blk_body(j, carry):
      t = wid + NW * j
      b = lax.div(t, K)
      k = lax.rem(t, K)
      fkv = plsc.load_gather(flip_v, [jnp.full((L,), k, dtype=jnp.int32)])
      fk = jnp.max(fkv)

      def in_copy(s):
        y0, ys = YSPLITS[s]
        return (in_hbm.at[b, fk, pl.ds(y0, ys)], ibuf.at[pl.ds(y0, ys)],
                isems[s])

      def out_copy(s):
        y0, ys = YSPLITS[s]
        return (obuf.at[pl.ds(y0, ys)], out_hbm.at[b, k, pl.ds(y0, ys)],
                osems[s])

      pltpu.async_copy(*in_copy(0))
      for s in range(NSPLIT):
        if s + 1 < NSPLIT:
          pltpu.async_copy(*in_copy(s + 1))
        pltpu.make_async_copy(*in_copy(s)).wait()

        @pl.when(j > 0)
        def _drain():
          pltpu.make_async_copy(*out_copy(s)).wait()

        y0, ys = YSPLITS[s]

        def row_body(y, rcarry):
          for c in range(c_dim):
            neg = signed and c == 0
            for jj in range(8):
              src = 105 - L * jj if jj < 7 else 0
              dst = L * jj if jj < 7 else 105
              v = lax.rev(ibuf[y, c, pl.ds(src, L)], (0,))
              if neg:
                v = -v
              obuf[y, c, pl.ds(dst, L)] = v
          return rcarry

        lax.fori_loop(y0, y0 + ys, row_body, 0)
        pltpu.async_copy(*out_copy(s))
      return carry

    lax.fori_loop(0, nb, blk_body, 0)

    @pl.when(nb > 0)
    def _final_drain():
      for s in range(NSPLIT):
        y0, ys = YSPLITS[s]
        pltpu.make_async_copy(obuf.at[pl.ds(y0, ys)],
                              out_hbm.at[0, 0, pl.ds(y0, ys)],
                              osems[s]).wait()

  do_field(f0_hbm, o0_hbm, i0, o0, 1, False)
  do_field(f1_hbm, o1_hbm, i1, o1, 2, True)


@jax.jit
def kernel(field0, field1, flip_indices):
  mesh = plsc.VectorSubcoreMesh(core_axis_name="c", subcore_axis_name="s",
                                num_cores=NC, num_subcores=NS)
  fn = pl.kernel(
      _body,
      out_type=[
          jax.ShapeDtypeStruct((B, K, W, 1, W), jnp.float32),
          jax.ShapeDtypeStruct((B, K, W, 2, W), jnp.float32),
      ],
      mesh=mesh,
      compiler_params=pltpu.CompilerParams(needs_layout_passes=False),
      scratch_types=[
          pltpu.VMEM((K,), jnp.int32),
          pltpu.VMEM((W, 1, W), jnp.float32),
          pltpu.VMEM((W, 1, W), jnp.float32),
          pltpu.VMEM((W, 2, W), jnp.float32),
          pltpu.VMEM((W, 2, W), jnp.float32),
      ] + [pltpu.SemaphoreType.DMA] * (2 * NSPLIT),
  )
  f0t = jnp.transpose(field0, (0, 1, 3, 2, 4))
  f1t = jnp.transpose(field1, (0, 1, 3, 2, 4))
  o0t, o1t = fn(f0t, f1t, flip_indices)
  return (jnp.transpose(o0t, (0, 1, 3, 2, 4)),
          jnp.transpose(o1t, (0, 1, 3, 2, 4)))
